# baseline (speedup 1.0000x reference)
import functools

import jax
import jax.numpy as jnp
from jax import lax
from jax.experimental import pallas as pl
from jax.experimental.pallas import tpu as pltpu
from jax.experimental.pallas import tpu_sc as plsc

VOCAB = 110000
EMB = 64
H = 128
NCLS = 20
B = 1024
L = 200

NSEG = 2
LSEG = L // NSEG
SEGTOT = B * LSEG
NW = 32
CH = 128
HCH = CH // 2
NCH = SEGTOT // (NW * CH)
NCHP = 32

LT = 10
NLC = LSEG // LT
BP = B // 2
DP = 2 * EMB
HP = 2 * H


def _sc_gather_body(table_hbm, idx_hbm, out_hbm,
                    idx_v, ra0, rb0, ra1, rb1, sem_0, sem_1):
    wid = lax.axis_index("s") * 2 + lax.axis_index("c")
    pltpu.sync_copy(idx_hbm.at[wid], idx_v)
    base = wid * NCH

    def fire(j, ra, rb, sem):
        pltpu.async_copy(table_hbm.at[idx_v.at[j, pl.ds(0, HCH)]], ra, sem)
        pltpu.async_copy(table_hbm.at[idx_v.at[j, pl.ds(HCH, HCH)]], rb, sem)

    def drain_scatter(j, ra, rb, sem):
        pltpu.make_async_copy(table_hbm.at[idx_v.at[0, pl.ds(0, HCH)]],
                              ra, sem).wait()
        pltpu.make_async_copy(table_hbm.at[idx_v.at[0, pl.ds(0, HCH)]],
                              rb, sem).wait()
        r0 = (base + j) * HCH
        pltpu.sync_copy(ra, out_hbm.at[pl.ds(r0, HCH), pl.ds(0, EMB)])
        pltpu.sync_copy(rb, out_hbm.at[pl.ds(r0, HCH), pl.ds(EMB, EMB)])

    fire(0, ra0, rb0, sem_0)

    def body(i, _):
        a = 2 * i
        fire(a + 1, ra1, rb1, sem_1)
        drain_scatter(a, ra0, rb0, sem_0)
        fire(a + 2, ra0, rb0, sem_0)
        drain_scatter(a + 1, ra1, rb1, sem_1)
        return 0

    lax.fori_loop(0, NCH // 2, body, 0)
    drain_scatter(NCH - 1, ra0, rb0, sem_0)


@functools.cache
def _sc_gather():
    return pl.kernel(
        _sc_gather_body,
        out_type=jax.ShapeDtypeStruct((SEGTOT // 2, DP), jnp.float32),
        mesh=plsc.VectorSubcoreMesh(core_axis_name="c", subcore_axis_name="s"),
        scratch_types=[
            pltpu.VMEM((NCHP, CH), jnp.int32),
            pltpu.VMEM((HCH, EMB), jnp.float32),
            pltpu.VMEM((HCH, EMB), jnp.float32),
            pltpu.VMEM((HCH, EMB), jnp.float32),
            pltpu.VMEM((HCH, EMB), jnp.float32),
            pltpu.SemaphoreType.DMA,
            pltpu.SemaphoreType.DMA,
        ],
        compiler_params=pltpu.CompilerParams(use_tc_tiling_on_sc=False),
    )


def _rnn_body(e_ref, h1in_ref, h2in_ref, w1_ref, wh1_ref, w2_ref, wh2_ref,
              fct_ref, b1_ref, b2_ref, fcb_ref,
              out_ref, h1out_ref, h2out_ref, h1_ref, h2_ref):
    lc = pl.program_id(0)

    @pl.when(lc == 0)
    def _():
        h1_ref[...] = h1in_ref[...]
        h2_ref[...] = h2in_ref[...]

    xp1 = jnp.dot(e_ref[...], w1_ref[...], preferred_element_type=jnp.float32)
    xp1 = xp1 + b1_ref[...]

    h1 = h1_ref[...]
    h2 = h2_ref[...]
    b2 = b2_ref[...]
    for t in range(LT):
        h1 = jnp.tanh(
            xp1[t * BP:(t + 1) * BP]
            + jnp.dot(h1, wh1_ref[...], preferred_element_type=jnp.float32))
        h2 = jnp.tanh(
            jnp.dot(h1, w2_ref[...], preferred_element_type=jnp.float32)
            + b2
            + jnp.dot(h2, wh2_ref[...], preferred_element_type=jnp.float32))
    h1_ref[...] = h1
    h2_ref[...] = h2

    @pl.when(lc == NLC - 1)
    def _():
        h1out_ref[...] = h1
        h2out_ref[...] = h2
        out_ref[...] = (
            jnp.dot(h2, fct_ref[...], preferred_element_type=jnp.float32)
            + fcb_ref[...])


_rnn_call = pl.pallas_call(
    _rnn_body,
    grid=(NLC,),
    in_specs=[
        pl.BlockSpec((LT * BP, DP), lambda l: (l, 0)),
        pl.BlockSpec((BP, HP), lambda l: (0, 0)),
        pl.BlockSpec((BP, HP), lambda l: (0, 0)),
        pl.BlockSpec((DP, HP), lambda l: (0, 0)),
        pl.BlockSpec((HP, HP), lambda l: (0, 0)),
        pl.BlockSpec((HP, HP), lambda l: (0, 0)),
        pl.BlockSpec((HP, HP), lambda l: (0, 0)),
        pl.BlockSpec((HP, 2 * NCLS), lambda l: (0, 0)),
        pl.BlockSpec((1, HP), lambda l: (0, 0)),
        pl.BlockSpec((1, HP), lambda l: (0, 0)),
        pl.BlockSpec((1, 2 * NCLS), lambda l: (0, 0)),
    ],
    out_specs=[
        pl.BlockSpec((BP, 2 * NCLS), lambda l: (0, 0)),
        pl.BlockSpec((BP, HP), lambda l: (0, 0)),
        pl.BlockSpec((BP, HP), lambda l: (0, 0)),
    ],
    out_shape=[
        jax.ShapeDtypeStruct((BP, 2 * NCLS), jnp.float32),
        jax.ShapeDtypeStruct((BP, HP), jnp.float32),
        jax.ShapeDtypeStruct((BP, HP), jnp.float32),
    ],
    scratch_shapes=[
        pltpu.VMEM((BP, HP), jnp.float32),
        pltpu.VMEM((BP, HP), jnp.float32),
    ],
    compiler_params=pltpu.CompilerParams(
        dimension_semantics=("arbitrary",)),
)


def _blkdiag(a):
    m, n = a.shape
    z = jnp.zeros((m, n), a.dtype)
    return jnp.concatenate(
        [jnp.concatenate([a, z], axis=1), jnp.concatenate([z, a], axis=1)],
        axis=0)


def kernel(x, emb, w_ih1, w_hh1, b_ih1, b_hh1,
           w_ih2, w_hh2, b_ih2, b_hh2, fc_w, fc_b):
    idxp = (x.T.astype(jnp.int32)
            .reshape(NSEG, NW, NCH, HCH, 2)
            .transpose(0, 1, 2, 4, 3)
            .reshape(NSEG, NW, NCH, CH))
    idxp = jnp.pad(idxp, ((0, 0), (0, 0), (0, NCHP - NCH), (0, 0)))
    es = [_sc_gather()(emb, idxp[s]) for s in range(NSEG)]

    w1 = _blkdiag(w_ih1.T)
    wh1 = _blkdiag(w_hh1.T)
    w2 = _blkdiag(w_ih2.T)
    wh2 = _blkdiag(w_hh2.T)
    fct = _blkdiag(fc_w.T)
    b1 = jnp.concatenate([b_ih1 + b_hh1] * 2)[None, :]
    b2 = jnp.concatenate([b_ih2 + b_hh2] * 2)[None, :]
    fcb = jnp.concatenate([fc_b] * 2)[None, :]

    h1 = jnp.zeros((BP, HP), jnp.float32)
    h2 = jnp.zeros((BP, HP), jnp.float32)
    out = None
    for s in range(NSEG):
        out, h1, h2 = _rnn_call(es[s], h1, h2, w1, wh1, w2, wh2,
                                fct, b1, b2, fcb)
    return out.reshape(B, NCLS)


# speedup vs baseline: 1.2082x; 1.2082x over previous
import functools

import jax
import jax.numpy as jnp
from jax import lax
from jax.experimental import pallas as pl
from jax.experimental.pallas import tpu as pltpu
from jax.experimental.pallas import tpu_sc as plsc

VOCAB = 110000
EMB = 64
H = 128
NCLS = 20
B = 1024
L = 200

NSEG = 2
LSEG = L // NSEG
SEGTOT = B * LSEG
NW = 32
CH = 128
NCH = SEGTOT // (NW * CH)
NCHP = 32

BM = 5000
NMC = VOCAB // BM

LT = 10
NLC = LSEG // LT


def _proj_body(e_ref, w_ref, b_ref, p_ref):
    p_ref[...] = (
        jnp.dot(e_ref[...], w_ref[...], preferred_element_type=jnp.float32)
        + b_ref[...])


_proj_call = pl.pallas_call(
    _proj_body,
    grid=(NMC,),
    in_specs=[
        pl.BlockSpec((BM, EMB), lambda i: (i, 0)),
        pl.BlockSpec((EMB, H), lambda i: (0, 0)),
        pl.BlockSpec((1, H), lambda i: (0, 0)),
    ],
    out_specs=pl.BlockSpec((BM, H), lambda i: (i, 0)),
    out_shape=jax.ShapeDtypeStruct((VOCAB, H), jnp.float32),
)


def _sc_gather_body(table_hbm, idx_hbm, out_hbm,
                    idx_v, rows_0, rows_1, sem_0, sem_1):
    wid = lax.axis_index("s") * 2 + lax.axis_index("c")
    pltpu.sync_copy(idx_hbm.at[wid], idx_v)
    base = wid * NCH

    def fire(j, buf, sem):
        return pltpu.async_copy(table_hbm.at[idx_v.at[j]], buf, sem)

    def drain_scatter(j, buf, sem):
        pltpu.make_async_copy(table_hbm.at[idx_v.at[0]], buf, sem).wait()
        pltpu.sync_copy(buf, out_hbm.at[pl.ds((base + j) * CH, CH)])

    fire(0, rows_0, sem_0)

    def body(i, _):
        a = 2 * i
        fire(a + 1, rows_1, sem_1)
        drain_scatter(a, rows_0, sem_0)
        fire(a + 2, rows_0, sem_0)
        drain_scatter(a + 1, rows_1, sem_1)
        return 0

    lax.fori_loop(0, NCH // 2, body, 0)
    drain_scatter(NCH - 1, rows_0, sem_0)


@functools.cache
def _sc_gather():
    return pl.kernel(
        _sc_gather_body,
        out_type=jax.ShapeDtypeStruct((SEGTOT, H), jnp.float32),
        mesh=plsc.VectorSubcoreMesh(core_axis_name="c", subcore_axis_name="s"),
        scratch_types=[
            pltpu.VMEM((NCHP, CH), jnp.int32),
            pltpu.VMEM((CH, H), jnp.float32),
            pltpu.VMEM((CH, H), jnp.float32),
            pltpu.SemaphoreType.DMA,
            pltpu.SemaphoreType.DMA,
        ],
        compiler_params=pltpu.CompilerParams(use_tc_tiling_on_sc=False),
    )


def _rnn_body(xp1_ref, h1in_ref, h2in_ref, wh1_ref, w2_ref, wh2_ref, fct_ref,
              b2_ref, fcb_ref, out_ref, h1out_ref, h2out_ref,
              h1_ref, h2_ref):
    lc = pl.program_id(0)

    @pl.when(lc == 0)
    def _():
        h1_ref[...] = h1in_ref[...]
        h2_ref[...] = h2in_ref[...]

    h1 = h1_ref[...]
    h2 = h2_ref[...]
    b2 = b2_ref[...]
    for t in range(LT):
        h1 = jnp.tanh(
            xp1_ref[t * B:(t + 1) * B]
            + jnp.dot(h1, wh1_ref[...], preferred_element_type=jnp.float32))
        h2 = jnp.tanh(
            jnp.dot(h1, w2_ref[...], preferred_element_type=jnp.float32)
            + b2
            + jnp.dot(h2, wh2_ref[...], preferred_element_type=jnp.float32))
    h1_ref[...] = h1
    h2_ref[...] = h2

    @pl.when(lc == NLC - 1)
    def _():
        h1out_ref[...] = h1
        h2out_ref[...] = h2
        out_ref[...] = (
            jnp.dot(h2, fct_ref[...], preferred_element_type=jnp.float32)
            + fcb_ref[...])


_rnn_call = pl.pallas_call(
    _rnn_body,
    grid=(NLC,),
    in_specs=[
        pl.BlockSpec((LT * B, H), lambda l: (l, 0)),
        pl.BlockSpec((B, H), lambda l: (0, 0)),
        pl.BlockSpec((B, H), lambda l: (0, 0)),
        pl.BlockSpec((H, H), lambda l: (0, 0)),
        pl.BlockSpec((H, H), lambda l: (0, 0)),
        pl.BlockSpec((H, H), lambda l: (0, 0)),
        pl.BlockSpec((H, NCLS), lambda l: (0, 0)),
        pl.BlockSpec((1, H), lambda l: (0, 0)),
        pl.BlockSpec((1, NCLS), lambda l: (0, 0)),
    ],
    out_specs=[
        pl.BlockSpec((B, NCLS), lambda l: (0, 0)),
        pl.BlockSpec((B, H), lambda l: (0, 0)),
        pl.BlockSpec((B, H), lambda l: (0, 0)),
    ],
    out_shape=[
        jax.ShapeDtypeStruct((B, NCLS), jnp.float32),
        jax.ShapeDtypeStruct((B, H), jnp.float32),
        jax.ShapeDtypeStruct((B, H), jnp.float32),
    ],
    scratch_shapes=[
        pltpu.VMEM((B, H), jnp.float32),
        pltpu.VMEM((B, H), jnp.float32),
    ],
    compiler_params=pltpu.CompilerParams(
        dimension_semantics=("arbitrary",)),
)


def kernel(x, emb, w_ih1, w_hh1, b_ih1, b_hh1,
           w_ih2, w_hh2, b_ih2, b_hh2, fc_w, fc_b):
    p = _proj_call(emb, w_ih1.T, (b_ih1 + b_hh1)[None, :])

    idx4 = jnp.pad(x.T.reshape(NSEG, NW, NCH, CH).astype(jnp.int32),
                   ((0, 0), (0, 0), (0, NCHP - NCH), (0, 0)))
    xps = [_sc_gather()(p, idx4[s]) for s in range(NSEG)]

    wh1 = w_hh1.T
    w2 = w_ih2.T
    wh2 = w_hh2.T
    fct = fc_w.T
    b2 = (b_ih2 + b_hh2)[None, :]
    fcb = fc_b[None, :]
    h1 = jnp.zeros((B, H), jnp.float32)
    h2 = jnp.zeros((B, H), jnp.float32)
    out = None
    for s in range(NSEG):
        out, h1, h2 = _rnn_call(xps[s], h1, h2, wh1, w2, wh2, fct, b2, fcb)
    return out
